# baseline (speedup 1.0000x reference)
import functools

import jax
import jax.numpy as jnp
import numpy as np
from jax import lax
from jax.experimental import pallas as pl
from jax.experimental.pallas import tpu as pltpu
from jax.experimental.pallas import tpu_sc as plsc

_NS = 16
_NV = 4
_NORM = 1.0 / np.sqrt(np.float32(_NS))
_CHUNK = 128
_NW = 32


def _build_constants():
    R = np.kron(np.eye(_NS, dtype=np.float32), np.ones((1, _NS), np.float32))
    T = np.kron(np.ones((1, _NS), np.float32), np.eye(_NS, dtype=np.float32))
    Q = np.zeros((32, 32), np.float32)
    for j in range(_NS):
        Q[j, j] = 1.0
    for c in range(_NV):
        for d in range(3):
            Q[_NS + c, _NS + 3 * c + d] = 1.0
    S = np.zeros((4, 32), np.float32)
    S[0, :_NS] = 1.0
    for c in range(_NV):
        for d in range(3):
            S[1 + d, _NS + 3 * c + d] = 1.0
    return R, T, Q, S


def _sc_gather(node_attr, dst2, e_pad):
    n_rows = dst2.shape[0]
    cpt = n_rows // _NW
    ept = cpt * _CHUNK
    mesh = plsc.VectorSubcoreMesh(core_axis_name="c", subcore_axis_name="s")

    @functools.partial(
        pl.kernel,
        mesh=mesh,
        out_type=jax.ShapeDtypeStruct((e_pad, _NS), jnp.float32),
        scratch_types=[
            pltpu.VMEM((cpt, _CHUNK), jnp.int32),
            pltpu.VMEM((ept, _NS), jnp.float32),
            pltpu.SemaphoreType.DMA,
        ],
        compiler_params=pltpu.CompilerParams(use_tc_tiling_on_sc=False),
    )
    def k(node_hbm, idx_hbm, out_hbm, idx_v, buf, sem):
        wid = lax.axis_index("s") * 2 + lax.axis_index("c")
        row0 = wid * cpt
        pltpu.sync_copy(idx_hbm.at[pl.ds(row0, cpt), :], idx_v)

        grp = 8

        def fire(g, carry):
            cps = []
            for t in range(grp):
                j = g * grp + t
                cps.append(pltpu.async_copy(
                    node_hbm.at[idx_v.at[j]],
                    buf.at[pl.ds(j * _CHUNK, _CHUNK), :], sem))
            for cp in cps:
                cp.wait()
            return carry

        lax.fori_loop(0, cpt // grp, fire, 0)
        pltpu.sync_copy(buf, out_hbm.at[pl.ds(row0 * _CHUNK, ept), :])

    return k(node_attr, dst2)


def _sc_scatter(tp32, src2, zinit, n_pad):
    n_rows = src2.shape[0]
    cpt = n_rows // _NW
    rpt = n_pad // 16
    nph = 4
    q = cpt // nph
    chunk_bytes = _CHUNK * 32 * 4
    mesh = plsc.VectorSubcoreMesh(core_axis_name="c", subcore_axis_name="s")

    @functools.partial(
        pl.kernel,
        mesh=mesh,
        out_type=jax.ShapeDtypeStruct((2, n_pad, 32), jnp.float32),
        scratch_types=[
            pltpu.VMEM((cpt, _CHUNK), jnp.int32),
            pltpu.VMEM(((cpt // 4) * _CHUNK, 32), jnp.float32),
            pltpu.VMEM(((cpt // 4) * _CHUNK, 32), jnp.float32),
            pltpu.VMEM_SHARED((n_pad, 32), jnp.float32),
            pltpu.SemaphoreType.DMA,
            pltpu.SemaphoreType.DMA,
        ],
        compiler_params=pltpu.CompilerParams(use_tc_tiling_on_sc=False),
    )
    def k(tp_hbm, idx_hbm, z_hbm, out_hbm, idx_v, buf_a, buf_b, acc, sem_ld, sem_sc):
        c = lax.axis_index("c")
        s = lax.axis_index("s")
        wid = s * 2 + c
        pltpu.sync_copy(z_hbm.at[pl.ds(s * rpt, rpt), :], acc.at[pl.ds(s * rpt, rpt), :])
        plsc.subcore_barrier()
        row0 = wid * cpt
        pltpu.sync_copy(idx_hbm.at[pl.ds(row0, cpt), :], idx_v)

        bufs = [buf_a, buf_b]
        cps = [pltpu.async_copy(tp_hbm.at[pl.ds(row0 * _CHUNK, q * _CHUNK), :],
                                buf_a, sem_ld)]
        for p in range(nph):
            buf = bufs[p % 2]
            if p + 1 < nph:
                cps.append(pltpu.async_copy(
                    tp_hbm.at[pl.ds((row0 + (p + 1) * q) * _CHUNK, q * _CHUNK), :],
                    bufs[(p + 1) % 2], sem_ld))
            cps[p].wait()

            def scat(j, carry, p=p, buf=buf):
                pltpu.sync_copy(buf.at[pl.ds(j * _CHUNK, _CHUNK), :],
                                acc.at[idx_v.at[p * q + j]], add=True)
                return carry

            lax.fori_loop(0, q, scat, 0)
        plsc.subcore_barrier()
        pltpu.sync_copy(acc.at[pl.ds(s * rpt, rpt), :],
                        out_hbm.at[c, pl.ds(s * rpt, rpt), :])

    return k(tp32, src2, zinit)


def _tc_dense(ea, x, sh, W1, b1, R, T, Mq, Bq, S, e_pad):
    bp = 200
    grid = (ea.shape[0] // bp,)

    def body(ea_ref, x2_ref, sh_ref, w1_ref, b1_ref, r_ref, t_ref, m_ref,
             b_ref, s_ref, out_ref):
        f32 = jnp.float32
        x2 = x2_ref[...]
        h = jnp.maximum(
            jnp.dot(ea_ref[...], w1_ref[...], preferred_element_type=f32)
            + b1_ref[...], 0.0)
        hr = jnp.dot(h, r_ref[...], preferred_element_type=f32)
        xt = jnp.dot(x2, t_ref[...], preferred_element_type=f32)
        base = (jnp.dot(hr * xt, m_ref[...], preferred_element_type=f32)
                + jnp.dot(x2, b_ref[...], preferred_element_type=f32))
        scale = jnp.dot(sh_ref[...], s_ref[...], preferred_element_type=f32)
        lane = lax.broadcasted_iota(jnp.int32, (bp, 256), 1)
        tp = jnp.where(lane % 32 == 28, 1.0, base * scale)
        out_ref[0] = tp[:, :128]
        out_ref[1] = tp[:, 128:]

    full = lambda shape: pl.BlockSpec(shape, lambda i: (0, 0))
    return pl.pallas_call(
        body,
        grid=grid,
        in_specs=[
            pl.BlockSpec((bp, 128), lambda i: (i, 0)),
            pl.BlockSpec((bp, 128), lambda i: (i, 0)),
            pl.BlockSpec((bp, 32), lambda i: (i, 0)),
            full((128, 128)),
            full((1, 128)),
            full((128, 2048)),
            full((128, 2048)),
            full((2048, 256)),
            full((128, 256)),
            full((32, 256)),
        ],
        out_specs=pl.BlockSpec((2, bp, 128), lambda i: (0, i, 0)),
        out_shape=jax.ShapeDtypeStruct((2, e_pad // 8, 128), jnp.float32),
        compiler_params=pltpu.CompilerParams(
            dimension_semantics=("arbitrary",)),
    )(ea, x, sh, W1, b1, R, T, Mq, Bq, S)


def _tc_epilogue(p0, p1, node_attr):
    n, ns = node_attr.shape
    out_w = _NS + 3 * _NV

    def body(p0_ref, p1_ref, na_ref, out_ref):
        s = p0_ref[...] + p1_ref[...]
        cnt = jnp.maximum(s[:, 28:29], 1.0)
        pad = jnp.concatenate(
            [na_ref[...], jnp.zeros((n, out_w - ns), jnp.float32)], axis=1)
        out_ref[...] = s[:, :out_w] / cnt + pad

    return pl.pallas_call(
        body,
        out_shape=jax.ShapeDtypeStruct((n, out_w), jnp.float32),
    )(p0, p1, node_attr)


def kernel(node_attr, edge_index, edge_attr, edge_sh, W1, b1, W2, b2):
    n, ns = node_attr.shape
    e = edge_attr.shape[0]
    e_pad = ((e + _NW * _CHUNK - 1) // (_NW * _CHUNK)) * (_NW * _CHUNK)
    n_pad = ((n + 1 + 15) // 16) * 16
    ep = e_pad - e

    edge_dst = jnp.pad(edge_index[1].astype(jnp.int32), (0, ep))
    edge_src = jnp.pad(edge_index[0].astype(jnp.int32), (0, ep),
                       constant_values=n)
    dst2 = edge_dst.reshape(e_pad // _CHUNK, _CHUNK)
    esp = edge_src.reshape(e_pad // 8, 2, 4)
    src_perm = jnp.concatenate(
        [esp[:, 0, :].reshape(-1), esp[:, 1, :].reshape(-1)])
    src2 = src_perm.reshape(e_pad // _CHUNK, _CHUNK)

    R, T, Q, S = _build_constants()
    M0 = W2[:, :_NS * _NS].reshape(_NS, _NS, _NS).reshape(_NS * _NS, _NS)
    M1 = W2[:, _NS * _NS:].reshape(_NS, _NS, _NV).reshape(_NS * _NS, _NV)
    M32 = jnp.concatenate([M0, M1, jnp.zeros((_NS * _NS, 12), jnp.float32)], 1)
    B0 = b2[:_NS * _NS].reshape(_NS, _NS)
    B1 = b2[_NS * _NS:].reshape(_NS, _NV)
    B32 = jnp.concatenate([B0, B1, jnp.zeros((_NS, 12), jnp.float32)], 1)
    Mq = (M32 @ Q) * _NORM
    Bq = (B32 @ Q) * _NORM

    i8 = jnp.eye(8, dtype=jnp.float32)
    i8np = np.eye(8, dtype=np.float32)
    W1d8 = jnp.kron(i8, W1)
    b1t = jnp.tile(b1, 8).reshape(1, 128)
    R8 = jnp.asarray(np.kron(i8np, R))
    T8 = jnp.asarray(np.kron(i8np, T))
    M8 = jnp.kron(i8, Mq)
    B8 = jnp.kron(i8, Bq)
    S8 = jnp.asarray(np.kron(i8np, S))

    x = _sc_gather(node_attr, dst2, e_pad)
    tp2 = _tc_dense(edge_attr.reshape(e // 8, 128),
                    x.reshape(e_pad // 8, 128),
                    edge_sh.reshape(e // 8, 32),
                    W1d8, b1t, R8, T8, M8, B8, S8, e_pad)
    zinit = jnp.zeros((n_pad, 32), jnp.float32)
    partials = _sc_scatter(tp2.reshape(e_pad, 32), src2, zinit, n_pad)
    out = _tc_epilogue(partials[0, :n, :], partials[1, :n, :], node_attr)
    return out


# speedup vs baseline: 1.2472x; 1.2472x over previous
import functools

import jax
import jax.numpy as jnp
import numpy as np
from jax import lax
from jax.experimental import pallas as pl
from jax.experimental.pallas import tpu as pltpu
from jax.experimental.pallas import tpu_sc as plsc

_NS = 16
_NV = 4
_NORM = 1.0 / np.sqrt(np.float32(_NS))
_CHUNK = 128
_NW = 32


def _build_constants():
    R = np.kron(np.eye(_NS, dtype=np.float32), np.ones((1, _NS), np.float32))
    T = np.kron(np.ones((1, _NS), np.float32), np.eye(_NS, dtype=np.float32))
    Q = np.zeros((32, 32), np.float32)
    for j in range(_NS):
        Q[j, j] = 1.0
    for c in range(_NV):
        for d in range(3):
            Q[_NS + c, _NS + 3 * c + d] = 1.0
    S = np.zeros((4, 32), np.float32)
    S[0, :_NS] = 1.0
    for c in range(_NV):
        for d in range(3):
            S[1 + d, _NS + 3 * c + d] = 1.0
    return R, T, Q, S


def _sc_gather(node_attr, dst2, e_pad):
    n_rows = dst2.shape[0]
    cpt = n_rows // _NW
    ept = cpt * _CHUNK
    mesh = plsc.VectorSubcoreMesh(core_axis_name="c", subcore_axis_name="s")

    @functools.partial(
        pl.kernel,
        mesh=mesh,
        out_type=jax.ShapeDtypeStruct((e_pad, _NS), jnp.float32),
        scratch_types=[
            pltpu.VMEM((cpt, _CHUNK), jnp.int32),
            pltpu.VMEM((ept, _NS), jnp.float32),
            pltpu.SemaphoreType.DMA,
        ],
        compiler_params=pltpu.CompilerParams(use_tc_tiling_on_sc=False),
    )
    def k(node_hbm, idx_hbm, out_hbm, idx_v, buf, sem):
        wid = lax.axis_index("s") * 2 + lax.axis_index("c")
        row0 = wid * cpt
        pltpu.sync_copy(idx_hbm.at[pl.ds(row0, cpt), :], idx_v)

        grp = 8

        def fire(g, carry):
            cps = []
            for t in range(grp):
                j = g * grp + t
                cps.append(pltpu.async_copy(
                    node_hbm.at[idx_v.at[j]],
                    buf.at[pl.ds(j * _CHUNK, _CHUNK), :], sem))
            for cp in cps:
                cp.wait()
            return carry

        lax.fori_loop(0, cpt // grp, fire, 0)
        pltpu.sync_copy(buf, out_hbm.at[pl.ds(row0 * _CHUNK, ept), :])

    return k(node_attr, dst2)


def _sc_scatter(tp32, src2, zinit, n_pad):
    n_rows = src2.shape[0]
    cpt = n_rows // _NW
    rpt = n_pad // 16
    nph = 4
    q = cpt // nph
    chunk_bytes = _CHUNK * 32 * 4
    mesh = plsc.VectorSubcoreMesh(core_axis_name="c", subcore_axis_name="s")

    @functools.partial(
        pl.kernel,
        mesh=mesh,
        out_type=jax.ShapeDtypeStruct((2, n_pad, 32), jnp.float32),
        scratch_types=[
            pltpu.VMEM((cpt, _CHUNK), jnp.int32),
            pltpu.VMEM(((cpt // 4) * _CHUNK, 32), jnp.float32),
            pltpu.VMEM(((cpt // 4) * _CHUNK, 32), jnp.float32),
            pltpu.VMEM_SHARED((n_pad, 32), jnp.float32),
            pltpu.SemaphoreType.DMA,
            pltpu.SemaphoreType.DMA,
        ],
        compiler_params=pltpu.CompilerParams(use_tc_tiling_on_sc=False),
    )
    def k(tp_hbm, idx_hbm, z_hbm, out_hbm, idx_v, buf_a, buf_b, acc, sem_ld, sem_sc):
        c = lax.axis_index("c")
        s = lax.axis_index("s")
        wid = s * 2 + c
        pltpu.sync_copy(z_hbm.at[pl.ds(s * rpt, rpt), :], acc.at[pl.ds(s * rpt, rpt), :])
        plsc.subcore_barrier()
        row0 = wid * cpt
        pltpu.sync_copy(idx_hbm.at[pl.ds(row0, cpt), :], idx_v)

        bufs = [buf_a, buf_b]
        cps = [pltpu.async_copy(tp_hbm.at[pl.ds(row0 * _CHUNK, q * _CHUNK), :],
                                buf_a, sem_ld)]
        for p in range(nph):
            buf = bufs[p % 2]
            if p + 1 < nph:
                cps.append(pltpu.async_copy(
                    tp_hbm.at[pl.ds((row0 + (p + 1) * q) * _CHUNK, q * _CHUNK), :],
                    bufs[(p + 1) % 2], sem_ld))
            cps[p].wait()

            def scat(j, carry, p=p, buf=buf):
                pltpu.sync_copy(buf.at[pl.ds(j * _CHUNK, _CHUNK), :],
                                acc.at[idx_v.at[p * q + j]], add=True)
                return carry

            lax.fori_loop(0, q, scat, 0)
        plsc.subcore_barrier()
        pltpu.sync_copy(acc.at[pl.ds(s * rpt, rpt), :],
                        out_hbm.at[c, pl.ds(s * rpt, rpt), :])

    return k(tp32, src2, zinit)


def _tc_dense(ea, x, sh, W1, b1, R, T, Mq, Bq, S, e_pad):
    e = ea.shape[0]
    be = 1600
    grid = (e // be,)

    def body(ea_ref, x_ref, sh_ref, w1_ref, b1_ref, r_ref, t_ref, mq_ref,
             bq_ref, s_ref, out_ref):
        f32 = jnp.float32
        x = x_ref[...]
        h = jnp.maximum(
            jnp.dot(ea_ref[...], w1_ref[...], preferred_element_type=f32)
            + b1_ref[...], 0.0)
        hr = jnp.dot(h, r_ref[...], preferred_element_type=f32)
        xt = jnp.dot(x, t_ref[...], preferred_element_type=f32)
        base = (jnp.dot(hr * xt, mq_ref[...], preferred_element_type=f32)
                + jnp.dot(x, bq_ref[...], preferred_element_type=f32))
        scale = jnp.dot(sh_ref[...], s_ref[...], preferred_element_type=f32)
        lane = lax.broadcasted_iota(jnp.int32, (be, 32), 1)
        out_ref[...] = jnp.where(lane == 28, 1.0, base * scale)

    full = lambda shape: pl.BlockSpec(shape, lambda i: (0, 0))
    return pl.pallas_call(
        body,
        grid=grid,
        in_specs=[
            pl.BlockSpec((be, _NS), lambda i: (i, 0)),
            pl.BlockSpec((be, _NS), lambda i: (i, 0)),
            pl.BlockSpec((be, 4), lambda i: (i, 0)),
            full((_NS, _NS)),
            full((1, _NS)),
            full((_NS, 256)),
            full((_NS, 256)),
            full((256, 32)),
            full((_NS, 32)),
            full((4, 32)),
        ],
        out_specs=pl.BlockSpec((be, 32), lambda i: (i, 0)),
        out_shape=jax.ShapeDtypeStruct((e_pad, 32), jnp.float32),
        compiler_params=pltpu.CompilerParams(
            dimension_semantics=("arbitrary",)),
    )(ea, x, sh, W1, b1, R, T, Mq, Bq, S)


def _tc_epilogue(p0, p1, node_attr):
    n, ns = node_attr.shape
    out_w = _NS + 3 * _NV

    def body(p0_ref, p1_ref, na_ref, out_ref):
        s = p0_ref[...] + p1_ref[...]
        cnt = jnp.maximum(s[:, 28:29], 1.0)
        pad = jnp.concatenate(
            [na_ref[...], jnp.zeros((n, out_w - ns), jnp.float32)], axis=1)
        out_ref[...] = s[:, :out_w] / cnt + pad

    return pl.pallas_call(
        body,
        out_shape=jax.ShapeDtypeStruct((n, out_w), jnp.float32),
    )(p0, p1, node_attr)


def kernel(node_attr, edge_index, edge_attr, edge_sh, W1, b1, W2, b2):
    n, ns = node_attr.shape
    e = edge_attr.shape[0]
    e_pad = ((e + _NW * _CHUNK - 1) // (_NW * _CHUNK)) * (_NW * _CHUNK)
    n_pad = ((n + 1 + 15) // 16) * 16
    ep = e_pad - e

    edge_dst = jnp.pad(edge_index[1].astype(jnp.int32), (0, ep))
    edge_src = jnp.pad(edge_index[0].astype(jnp.int32), (0, ep),
                       constant_values=n)
    dst2 = edge_dst.reshape(e_pad // _CHUNK, _CHUNK)
    src2 = edge_src.reshape(e_pad // _CHUNK, _CHUNK)

    R, T, Q, S = _build_constants()
    M0 = W2[:, :_NS * _NS].reshape(_NS, _NS, _NS).reshape(_NS * _NS, _NS)
    M1 = W2[:, _NS * _NS:].reshape(_NS, _NS, _NV).reshape(_NS * _NS, _NV)
    M32 = jnp.concatenate([M0, M1, jnp.zeros((_NS * _NS, 12), jnp.float32)], 1)
    B0 = b2[:_NS * _NS].reshape(_NS, _NS)
    B1 = b2[_NS * _NS:].reshape(_NS, _NV)
    B32 = jnp.concatenate([B0, B1, jnp.zeros((_NS, 12), jnp.float32)], 1)
    Mq = (M32 @ Q) * _NORM
    Bq = (B32 @ Q) * _NORM

    x = _sc_gather(node_attr, dst2, e_pad)
    tp = _tc_dense(edge_attr, x, edge_sh, W1, b1.reshape(1, _NS),
                   jnp.asarray(R), jnp.asarray(T), Mq, Bq, jnp.asarray(S),
                   e_pad)
    zinit = jnp.zeros((n_pad, 32), jnp.float32)
    partials = _sc_scatter(tp, src2, zinit, n_pad)
    out = _tc_epilogue(partials[0, :n, :], partials[1, :n, :], node_attr)
    return out


# speedup vs baseline: 1.5810x; 1.2676x over previous
import functools

import jax
import jax.numpy as jnp
import numpy as np
from jax import lax
from jax.experimental import pallas as pl
from jax.experimental.pallas import tpu as pltpu
from jax.experimental.pallas import tpu_sc as plsc

_NS = 16
_NV = 4
_NORM = 1.0 / np.sqrt(np.float32(_NS))
_CHUNK = 128
_NW = 32


def _build_constants():
    R = np.kron(np.eye(_NS, dtype=np.float32), np.ones((1, _NS), np.float32))
    T = np.kron(np.ones((1, _NS), np.float32), np.eye(_NS, dtype=np.float32))
    Q = np.zeros((32, 32), np.float32)
    for j in range(_NS):
        Q[j, j] = 1.0
    for c in range(_NV):
        for d in range(3):
            Q[_NS + c, _NS + 3 * c + d] = 1.0
    S = np.zeros((4, 32), np.float32)
    S[0, :_NS] = 1.0
    for c in range(_NV):
        for d in range(3):
            S[1 + d, _NS + 3 * c + d] = 1.0
    return R, T, Q, S


def _sc_gather(node_attr, dst2, e_pad):
    n_rows = dst2.shape[0]
    cpt = n_rows // _NW
    ept = cpt * _CHUNK
    mesh = plsc.VectorSubcoreMesh(core_axis_name="c", subcore_axis_name="s")

    @functools.partial(
        pl.kernel,
        mesh=mesh,
        out_type=jax.ShapeDtypeStruct((e_pad, _NS), jnp.float32),
        scratch_types=[
            pltpu.VMEM((cpt, _CHUNK), jnp.int32),
            pltpu.VMEM((ept, _NS), jnp.float32),
            pltpu.SemaphoreType.DMA,
        ],
        compiler_params=pltpu.CompilerParams(use_tc_tiling_on_sc=False),
    )
    def k(node_hbm, idx_hbm, out_hbm, idx_v, buf, sem):
        wid = lax.axis_index("s") * 2 + lax.axis_index("c")
        row0 = wid * cpt
        pltpu.sync_copy(idx_hbm.at[pl.ds(row0, cpt), :], idx_v)

        grp = 8

        def fire(g, carry):
            cps = []
            for t in range(grp):
                j = g * grp + t
                cps.append(pltpu.async_copy(
                    node_hbm.at[idx_v.at[j]],
                    buf.at[pl.ds(j * _CHUNK, _CHUNK), :], sem))
            for cp in cps:
                cp.wait()
            return carry

        lax.fori_loop(0, cpt // grp, fire, 0)
        pltpu.sync_copy(buf, out_hbm.at[pl.ds(row0 * _CHUNK, ept), :])

    return k(node_attr, dst2)


def _sc_scatter(tp32, src2, zinit, n_pad):
    n_rows = src2.shape[0]
    cpt = n_rows // _NW
    rpt = n_pad // 16
    nph = 4
    q = cpt // nph
    chunk_bytes = _CHUNK * 32 * 4
    mesh = plsc.VectorSubcoreMesh(core_axis_name="c", subcore_axis_name="s")

    @functools.partial(
        pl.kernel,
        mesh=mesh,
        out_type=jax.ShapeDtypeStruct((2, n_pad, 32), jnp.float32),
        scratch_types=[
            pltpu.VMEM((cpt, _CHUNK), jnp.int32),
            pltpu.VMEM(((cpt // 4) * _CHUNK, 32), jnp.float32),
            pltpu.VMEM(((cpt // 4) * _CHUNK, 32), jnp.float32),
            pltpu.VMEM_SHARED((n_pad, 32), jnp.float32),
            pltpu.SemaphoreType.DMA,
            pltpu.SemaphoreType.DMA,
        ],
        compiler_params=pltpu.CompilerParams(use_tc_tiling_on_sc=False),
    )
    def k(tp_hbm, idx_hbm, z_hbm, out_hbm, idx_v, buf_a, buf_b, acc, sem_ld, sem_sc):
        c = lax.axis_index("c")
        s = lax.axis_index("s")
        wid = s * 2 + c
        pltpu.sync_copy(z_hbm.at[pl.ds(s * rpt, rpt), :], acc.at[pl.ds(s * rpt, rpt), :])
        plsc.subcore_barrier()
        row0 = wid * cpt
        pltpu.sync_copy(idx_hbm.at[pl.ds(row0, cpt), :], idx_v)

        bufs = [buf_a, buf_b]
        cps = [pltpu.async_copy(tp_hbm.at[pl.ds(row0 * _CHUNK, q * _CHUNK), :],
                                buf_a, sem_ld)]
        for p in range(nph):
            buf = bufs[p % 2]
            if p + 1 < nph:
                cps.append(pltpu.async_copy(
                    tp_hbm.at[pl.ds((row0 + (p + 1) * q) * _CHUNK, q * _CHUNK), :],
                    bufs[(p + 1) % 2], sem_ld))
            cps[p].wait()

            def scat(j, carry, p=p, buf=buf):
                pltpu.sync_copy(buf.at[pl.ds(j * _CHUNK, _CHUNK), :],
                                acc.at[idx_v.at[p * q + j]], add=True)
                return carry

            lax.fori_loop(0, q, scat, 0)
        plsc.subcore_barrier()
        pltpu.sync_copy(acc.at[pl.ds(s * rpt, rpt), :],
                        out_hbm.at[c, pl.ds(s * rpt, rpt), :])

    return k(tp32, src2, zinit)


def _tc_dense(eaT, x, shT, W1t, b1c, Rt, Tt, Mqt, Bqt, St, e_pad):
    e = eaT.shape[1]
    be = 2048
    grid = ((e + be - 1) // be,)

    def body(eaT_ref, x_ref, shT_ref, w1t_ref, b1c_ref, rt_ref, tt_ref,
             mqt_ref, bqt_ref, st_ref, out_ref):
        f32 = jnp.float32
        xT = x_ref[...].T
        hT = jnp.maximum(
            jnp.dot(w1t_ref[...], eaT_ref[...], preferred_element_type=f32)
            + b1c_ref[...], 0.0)
        hrT = jnp.dot(rt_ref[...], hT, preferred_element_type=f32)
        xtT = jnp.dot(tt_ref[...], xT, preferred_element_type=f32)
        baseT = (jnp.dot(mqt_ref[...], hrT * xtT, preferred_element_type=f32)
                 + jnp.dot(bqt_ref[...], xT, preferred_element_type=f32))
        scaleT = jnp.dot(st_ref[...], shT_ref[...], preferred_element_type=f32)
        row = lax.broadcasted_iota(jnp.int32, (32, be), 0)
        tpT = jnp.where(row == 28, 1.0, baseT * scaleT)
        out_ref[...] = tpT.T

    full = lambda shape: pl.BlockSpec(shape, lambda i: (0, 0))
    return pl.pallas_call(
        body,
        grid=grid,
        in_specs=[
            pl.BlockSpec((_NS, be), lambda i: (0, i)),
            pl.BlockSpec((be, _NS), lambda i: (i, 0)),
            pl.BlockSpec((4, be), lambda i: (0, i)),
            full((_NS, _NS)),
            full((_NS, 1)),
            full((256, _NS)),
            full((256, _NS)),
            full((32, 256)),
            full((32, _NS)),
            full((32, 4)),
        ],
        out_specs=pl.BlockSpec((be, 32), lambda i: (i, 0)),
        out_shape=jax.ShapeDtypeStruct((e_pad, 32), jnp.float32),
        compiler_params=pltpu.CompilerParams(
            dimension_semantics=("arbitrary",)),
    )(eaT, x, shT, W1t, b1c, Rt, Tt, Mqt, Bqt, St)


def _tc_epilogue(p0, p1, node_attr):
    n, ns = node_attr.shape
    out_w = _NS + 3 * _NV

    def body(p0_ref, p1_ref, na_ref, out_ref):
        s = p0_ref[...] + p1_ref[...]
        cnt = jnp.maximum(s[:, 28:29], 1.0)
        pad = jnp.concatenate(
            [na_ref[...], jnp.zeros((n, out_w - ns), jnp.float32)], axis=1)
        out_ref[...] = s[:, :out_w] / cnt + pad

    return pl.pallas_call(
        body,
        out_shape=jax.ShapeDtypeStruct((n, out_w), jnp.float32),
    )(p0, p1, node_attr)


def kernel(node_attr, edge_index, edge_attr, edge_sh, W1, b1, W2, b2):
    n, ns = node_attr.shape
    e = edge_attr.shape[0]
    e_pad = ((e + _NW * _CHUNK - 1) // (_NW * _CHUNK)) * (_NW * _CHUNK)
    n_pad = ((n + 1 + 15) // 16) * 16
    ep = e_pad - e

    edge_dst = jnp.pad(edge_index[1].astype(jnp.int32), (0, ep))
    edge_src = jnp.pad(edge_index[0].astype(jnp.int32), (0, ep),
                       constant_values=n)
    dst2 = edge_dst.reshape(e_pad // _CHUNK, _CHUNK)
    src2 = edge_src.reshape(e_pad // _CHUNK, _CHUNK)

    R, T, Q, S = _build_constants()
    M0 = W2[:, :_NS * _NS].reshape(_NS, _NS, _NS).reshape(_NS * _NS, _NS)
    M1 = W2[:, _NS * _NS:].reshape(_NS, _NS, _NV).reshape(_NS * _NS, _NV)
    M32 = jnp.concatenate([M0, M1, jnp.zeros((_NS * _NS, 12), jnp.float32)], 1)
    B0 = b2[:_NS * _NS].reshape(_NS, _NS)
    B1 = b2[_NS * _NS:].reshape(_NS, _NV)
    B32 = jnp.concatenate([B0, B1, jnp.zeros((_NS, 12), jnp.float32)], 1)
    Mq = (M32 @ Q) * _NORM
    Bq = (B32 @ Q) * _NORM

    x = _sc_gather(node_attr, dst2, e_pad)
    tp = _tc_dense(edge_attr.T, x, edge_sh.T, W1.T, b1.reshape(_NS, 1),
                   jnp.asarray(R.T.copy()), jnp.asarray(T.T.copy()),
                   Mq.T, Bq.T, jnp.asarray(S.T.copy()), e_pad)
    zinit = jnp.zeros((n_pad, 32), jnp.float32)
    partials = _sc_scatter(tp, src2, zinit, n_pad)
    out = _tc_epilogue(partials[0, :n, :], partials[1, :n, :], node_attr)
    return out


# speedup vs baseline: 1.5848x; 1.0024x over previous
import functools

import jax
import jax.numpy as jnp
import numpy as np
from jax import lax
from jax.experimental import pallas as pl
from jax.experimental.pallas import tpu as pltpu
from jax.experimental.pallas import tpu_sc as plsc

_NS = 16
_NV = 4
_NORM = 1.0 / np.sqrt(np.float32(_NS))
_CHUNK = 128
_NW = 32


def _build_constants():
    R = np.kron(np.eye(_NS, dtype=np.float32), np.ones((1, _NS), np.float32))
    T = np.kron(np.ones((1, _NS), np.float32), np.eye(_NS, dtype=np.float32))
    Q = np.zeros((32, 32), np.float32)
    for j in range(_NS):
        Q[j, j] = 1.0
    for c in range(_NV):
        for d in range(3):
            Q[_NS + c, _NS + 3 * c + d] = 1.0
    S = np.zeros((4, 32), np.float32)
    S[0, :_NS] = 1.0
    for c in range(_NV):
        for d in range(3):
            S[1 + d, _NS + 3 * c + d] = 1.0
    return R, T, Q, S


def _sc_gather(node_attr, dst2, e_pad):
    n_rows = dst2.shape[0]
    cpt = n_rows // _NW
    ept = cpt * _CHUNK
    mesh = plsc.VectorSubcoreMesh(core_axis_name="c", subcore_axis_name="s")

    @functools.partial(
        pl.kernel,
        mesh=mesh,
        out_type=jax.ShapeDtypeStruct((e_pad, _NS), jnp.float32),
        scratch_types=[
            pltpu.VMEM((cpt, _CHUNK), jnp.int32),
            pltpu.VMEM((ept, _NS), jnp.float32),
            pltpu.SemaphoreType.DMA,
        ],
        compiler_params=pltpu.CompilerParams(use_tc_tiling_on_sc=False),
    )
    def k(node_hbm, idx_hbm, out_hbm, idx_v, buf, sem):
        wid = lax.axis_index("s") * 2 + lax.axis_index("c")
        row0 = wid * cpt
        pltpu.sync_copy(idx_hbm.at[pl.ds(row0, cpt), :], idx_v)

        grp = 20

        def fire(g, carry):
            cps = []
            for t in range(grp):
                j = g * grp + t
                cps.append(pltpu.async_copy(
                    node_hbm.at[idx_v.at[j]],
                    buf.at[pl.ds(j * _CHUNK, _CHUNK), :], sem))
            for cp in cps:
                cp.wait()
            return carry

        lax.fori_loop(0, cpt // grp, fire, 0)
        pltpu.sync_copy(buf, out_hbm.at[pl.ds(row0 * _CHUNK, ept), :])

    return k(node_attr, dst2)


def _sc_scatter(tp32, src2, zinit, n_pad):
    n_rows = src2.shape[0]
    cpt = n_rows // _NW
    rpt = n_pad // 16
    nph = 4
    q = cpt // nph
    chunk_bytes = _CHUNK * 32 * 4
    mesh = plsc.VectorSubcoreMesh(core_axis_name="c", subcore_axis_name="s")

    @functools.partial(
        pl.kernel,
        mesh=mesh,
        out_type=jax.ShapeDtypeStruct((2, n_pad, 32), jnp.float32),
        scratch_types=[
            pltpu.VMEM((cpt, _CHUNK), jnp.int32),
            pltpu.VMEM(((cpt // 4) * _CHUNK, 32), jnp.float32),
            pltpu.VMEM(((cpt // 4) * _CHUNK, 32), jnp.float32),
            pltpu.VMEM_SHARED((n_pad, 32), jnp.float32),
            pltpu.SemaphoreType.DMA,
            pltpu.SemaphoreType.DMA,
        ],
        compiler_params=pltpu.CompilerParams(use_tc_tiling_on_sc=False),
    )
    def k(tp_hbm, idx_hbm, z_hbm, out_hbm, idx_v, buf_a, buf_b, acc, sem_ld, sem_sc):
        c = lax.axis_index("c")
        s = lax.axis_index("s")
        wid = s * 2 + c
        pltpu.sync_copy(z_hbm.at[pl.ds(s * rpt, rpt), :], acc.at[pl.ds(s * rpt, rpt), :])
        plsc.subcore_barrier()
        row0 = wid * cpt
        pltpu.sync_copy(idx_hbm.at[pl.ds(row0, cpt), :], idx_v)

        bufs = [buf_a, buf_b]
        cps = [pltpu.async_copy(tp_hbm.at[pl.ds(row0 * _CHUNK, q * _CHUNK), :],
                                buf_a, sem_ld)]
        for p in range(nph):
            buf = bufs[p % 2]
            if p + 1 < nph:
                cps.append(pltpu.async_copy(
                    tp_hbm.at[pl.ds((row0 + (p + 1) * q) * _CHUNK, q * _CHUNK), :],
                    bufs[(p + 1) % 2], sem_ld))
            cps[p].wait()

            def scat(j, carry, p=p, buf=buf):
                pltpu.sync_copy(buf.at[pl.ds(j * _CHUNK, _CHUNK), :],
                                acc.at[idx_v.at[p * q + j]], add=True)
                return carry

            lax.fori_loop(0, q, scat, 0)
        plsc.subcore_barrier()
        pltpu.sync_copy(acc.at[pl.ds(s * rpt, rpt), :],
                        out_hbm.at[c, pl.ds(s * rpt, rpt), :])

    return k(tp32, src2, zinit)


def _tc_dense(eaT, x, shT, W1t, b1c, Rt, Tt, Mqt, Bqt, St, e_pad):
    e = eaT.shape[1]
    be = 2048
    grid = ((e + be - 1) // be,)

    def body(eaT_ref, x_ref, shT_ref, w1t_ref, b1c_ref, rt_ref, tt_ref,
             mqt_ref, bqt_ref, st_ref, out_ref):
        f32 = jnp.float32
        xT = x_ref[...].T
        hT = jnp.maximum(
            jnp.dot(w1t_ref[...], eaT_ref[...], preferred_element_type=f32)
            + b1c_ref[...], 0.0)
        hrT = jnp.dot(rt_ref[...], hT, preferred_element_type=f32)
        xtT = jnp.dot(tt_ref[...], xT, preferred_element_type=f32)
        baseT = (jnp.dot(mqt_ref[...], hrT * xtT, preferred_element_type=f32)
                 + jnp.dot(bqt_ref[...], xT, preferred_element_type=f32))
        scaleT = jnp.dot(st_ref[...], shT_ref[...], preferred_element_type=f32)
        row = lax.broadcasted_iota(jnp.int32, (32, be), 0)
        tpT = jnp.where(row == 28, 1.0, baseT * scaleT)
        out_ref[...] = tpT.T

    full = lambda shape: pl.BlockSpec(shape, lambda i: (0, 0))
    return pl.pallas_call(
        body,
        grid=grid,
        in_specs=[
            pl.BlockSpec((_NS, be), lambda i: (0, i)),
            pl.BlockSpec((be, _NS), lambda i: (i, 0)),
            pl.BlockSpec((4, be), lambda i: (0, i)),
            full((_NS, _NS)),
            full((_NS, 1)),
            full((256, _NS)),
            full((256, _NS)),
            full((32, 256)),
            full((32, _NS)),
            full((32, 4)),
        ],
        out_specs=pl.BlockSpec((be, 32), lambda i: (i, 0)),
        out_shape=jax.ShapeDtypeStruct((e_pad, 32), jnp.float32),
        compiler_params=pltpu.CompilerParams(
            dimension_semantics=("arbitrary",)),
    )(eaT, x, shT, W1t, b1c, Rt, Tt, Mqt, Bqt, St)


def _tc_epilogue(p0, p1, node_attr):
    n, ns = node_attr.shape
    out_w = _NS + 3 * _NV

    def body(p0_ref, p1_ref, na_ref, out_ref):
        s = p0_ref[...] + p1_ref[...]
        cnt = jnp.maximum(s[:, 28:29], 1.0)
        pad = jnp.concatenate(
            [na_ref[...], jnp.zeros((n, out_w - ns), jnp.float32)], axis=1)
        out_ref[...] = s[:, :out_w] / cnt + pad

    return pl.pallas_call(
        body,
        out_shape=jax.ShapeDtypeStruct((n, out_w), jnp.float32),
    )(p0, p1, node_attr)


def kernel(node_attr, edge_index, edge_attr, edge_sh, W1, b1, W2, b2):
    n, ns = node_attr.shape
    e = edge_attr.shape[0]
    e_pad = ((e + _NW * _CHUNK - 1) // (_NW * _CHUNK)) * (_NW * _CHUNK)
    n_pad = ((n + 1 + 15) // 16) * 16
    ep = e_pad - e

    edge_dst = jnp.pad(edge_index[1].astype(jnp.int32), (0, ep))
    edge_src = jnp.pad(edge_index[0].astype(jnp.int32), (0, ep),
                       constant_values=n)
    dst2 = edge_dst.reshape(e_pad // _CHUNK, _CHUNK)
    src2 = edge_src.reshape(e_pad // _CHUNK, _CHUNK)

    R, T, Q, S = _build_constants()
    M0 = W2[:, :_NS * _NS].reshape(_NS, _NS, _NS).reshape(_NS * _NS, _NS)
    M1 = W2[:, _NS * _NS:].reshape(_NS, _NS, _NV).reshape(_NS * _NS, _NV)
    M32 = jnp.concatenate([M0, M1, jnp.zeros((_NS * _NS, 12), jnp.float32)], 1)
    B0 = b2[:_NS * _NS].reshape(_NS, _NS)
    B1 = b2[_NS * _NS:].reshape(_NS, _NV)
    B32 = jnp.concatenate([B0, B1, jnp.zeros((_NS, 12), jnp.float32)], 1)
    Mq = (M32 @ Q) * _NORM
    Bq = (B32 @ Q) * _NORM

    x = _sc_gather(node_attr, dst2, e_pad)
    tp = _tc_dense(edge_attr.T, x, edge_sh.T, W1.T, b1.reshape(_NS, 1),
                   jnp.asarray(R.T.copy()), jnp.asarray(T.T.copy()),
                   Mq.T, Bq.T, jnp.asarray(S.T.copy()), e_pad)
    zinit = jnp.zeros((n_pad, 32), jnp.float32)
    partials = _sc_scatter(tp, src2, zinit, n_pad)
    out = _tc_epilogue(partials[0, :n, :], partials[1, :n, :], node_attr)
    return out
